# baseline (speedup 1.0000x reference)
import functools

import jax
import jax.numpy as jnp
from jax import lax
from jax.experimental import pallas as pl
from jax.experimental.pallas import tpu as pltpu
from jax.experimental.pallas import tpu_sc as plsc

NC = 2
NS = 16
LANES = 16
CH = 128
W = 80
HD = 64
NBUF = 3

_DOT = dict(preferred_element_type=jnp.float32, precision=lax.Precision.HIGHEST)


def _padw(m, cols):
    r = m.shape[0]
    return jnp.concatenate(
        [m, jnp.zeros((r, W - m.shape[1]), jnp.float32)], axis=1)


def _bias_half(b, lo):
    return jnp.concatenate(
        [b[lo:lo + HD].reshape(1, HD), jnp.ones((1, 1), jnp.float32),
         jnp.zeros((1, W - HD - 1), jnp.float32)], axis=1)


def _tc_pre_body(x_ref, w1p0_ref, w1p1_ref, b1p0_ref, b1p1_ref, w2a_ref,
                 a_ref, bp_ref):
    x = x_ref[...]
    a_ref[0] = jnp.dot(x, w1p0_ref[...], **_DOT) + b1p0_ref[...]
    a_ref[1] = jnp.dot(x, w1p1_ref[...], **_DOT) + b1p1_ref[...]
    bp_ref[...] = jnp.dot(x, w2a_ref[...], **_DOT)


def _tc_mid_body(se_ref, wr0p0_ref, wr1p0_ref, wr0p1_ref, wr1p1_ref,
                 w2cp0_ref, w2cp1_ref, b2p0_ref, b2p1_ref, c_ref):
    de = jnp.sum(se_ref[0][:, HD:], axis=1)
    inv = 1.0 / jnp.maximum(de, 1.0)
    xe0 = se_ref[0][:, :HD] * inv[:, None]
    xe1 = se_ref[1][:, :HD] * inv[:, None]
    logde = jnp.log(de)[:, None]
    c_ref[0] = (jnp.dot(xe0, wr0p0_ref[...], **_DOT)
                + jnp.dot(xe1, wr1p0_ref[...], **_DOT)
                + logde * w2cp0_ref[...] + b2p0_ref[...])
    c_ref[1] = (jnp.dot(xe0, wr0p1_ref[...], **_DOT)
                + jnp.dot(xe1, wr1p1_ref[...], **_DOT)
                + logde * w2cp1_ref[...] + b2p1_ref[...])


def _tc_post_body(t_ref, bp_ref, x_ref, x0_ref, w3a0_ref, w3a1_ref, w3b_ref,
                  w3c_ref, w3d_ref, b31_ref, w32_ref, b32_ref, out_ref):
    dv = jnp.sum(t_ref[0][:, HD:], axis=1)
    inv = 1.0 / jnp.maximum(dv, 1.0)
    dvc = dv[:, None]
    bp = bp_ref[...]
    xv0 = (dvc * bp[:, :HD] + t_ref[0][:, :HD]) * inv[:, None]
    xv1 = (dvc * bp[:, HD:] + t_ref[1][:, :HD]) * inv[:, None]
    pre = (
        jnp.dot(xv0, w3a0_ref[...], **_DOT)
        + jnp.dot(xv1, w3a1_ref[...], **_DOT)
        + jnp.dot(x_ref[...], w3b_ref[...], **_DOT)
        + jnp.dot(x0_ref[...], w3c_ref[...], **_DOT)
        + jnp.log(dv)[:, None] * w3d_ref[...]
        + b31_ref[...]
    )
    h = jnp.maximum(pre, 0.0)
    out_ref[...] = jnp.dot(h, w32_ref[...], **_DOT) + b32_ref[...]


def _zero_rows(rows_v, n_rows, d):
    zero = jnp.zeros((LANES,), jnp.float32)

    def body(i, carry):
        for c in range(d // LANES):
            rows_v[i, pl.ds(c * LANES, LANES)] = zero
        return carry

    lax.fori_loop(0, n_rows, body, 0)


def _sc_segsum_body(n_chunks, gpw, acc_rows, table_hbm, gidx_hbm, sidx_hbm,
                    out_hbm, acc_sh, gidx, sidx, rows_v, semi,
                    semg0, semg1, semg2, sems0, sems1, sems2):
    cid = lax.axis_index("c")
    tid = lax.axis_index("s")
    semg = (semg0, semg1, semg2)
    sems = (sems0, sems1, sems2)
    wid = tid
    base = wid * gpw

    pltpu.async_copy(gidx_hbm.at[cid].at[pl.ds(base, gpw)], gidx, semi)
    pltpu.async_copy(sidx_hbm.at[pl.ds(base, gpw)], sidx, semi)

    _zero_rows(rows_v.at[0], CH, W)
    rows_per_tile = acc_rows // NS
    off = 0
    while off < rows_per_tile:
        step = min(CH, rows_per_tile - off)
        pltpu.sync_copy(rows_v.at[0].at[pl.ds(0, step)],
                        acc_sh.at[pl.ds(tid * rows_per_tile + off, step)])
        off += step
    pltpu.make_async_copy(gidx_hbm.at[cid].at[pl.ds(base, gpw)],
                          gidx, semi).wait()
    pltpu.make_async_copy(sidx_hbm.at[pl.ds(base, gpw)], sidx, semi).wait()
    plsc.subcore_barrier()

    def valid(g):
        return jnp.logical_and(g < gpw, base + g < n_chunks)

    def gather_start(g, b):
        @pl.when(valid(g))
        def _():
            pltpu.async_copy(table_hbm.at[gidx.at[g]], rows_v.at[b], semg[b])

    def gather_wait(g, b):
        @pl.when(valid(g))
        def _():
            pltpu.make_async_copy(table_hbm.at[gidx.at[g]],
                                  rows_v.at[b], semg[b]).wait()

    def scat_start(g, b):
        @pl.when(valid(g))
        def _():
            pltpu.async_copy(rows_v.at[b], acc_sh.at[sidx.at[g]],
                             sems[b], add=True)

    def scat_wait(g, b):
        @pl.when(jnp.logical_and(g >= 0, valid(g)))
        def _():
            gc = jnp.maximum(g, 0)
            pltpu.make_async_copy(rows_v.at[b], acc_sh.at[sidx.at[gc]],
                                  sems[b]).wait()

    gather_start(0, 0)

    def ring(gh, carry):
        for b in range(NBUF):
            g = gh * NBUF + b
            nb = (b + 1) % NBUF
            gather_wait(g, b)
            scat_start(g, b)
            scat_wait(g - (NBUF - 1), nb)
            gather_start(g + 1, nb)
        return carry

    lax.fori_loop(0, -(-(gpw + NBUF - 1) // NBUF), ring, 0)
    plsc.subcore_barrier()

    pltpu.sync_copy(acc_sh.at[pl.ds(tid * rows_per_tile, rows_per_tile)],
                    out_hbm.at[cid, pl.ds(tid * rows_per_tile, rows_per_tile)])


def _sc_segsum(table_flat, gidx3d, sidx2d, acc_rows, n_chunks):
    gpw = -(-n_chunks // NS)
    mesh = plsc.VectorSubcoreMesh(core_axis_name="c", subcore_axis_name="s")
    return pl.kernel(
        functools.partial(_sc_segsum_body, n_chunks, gpw, acc_rows),
        out_type=jax.ShapeDtypeStruct((NC, acc_rows, W), jnp.float32),
        mesh=mesh,
        scratch_types=[
            pltpu.VMEM_SHARED((acc_rows, W), jnp.float32),
            pltpu.VMEM((gpw, CH), jnp.int32),
            pltpu.VMEM((gpw, CH), jnp.int32),
            pltpu.VMEM((NBUF, CH, W), jnp.float32),
        ] + [pltpu.SemaphoreType.DMA] * (2 * NBUF + 1),
        compiler_params=pltpu.CompilerParams(use_tc_tiling_on_sc=False),
    )(table_flat, gidx3d, sidx2d)


def kernel(X, vertex, edges, X0, W1_w, W1_b, W2_w, W2_b, W3_w1, W3_b1,
           W3_w2, W3_b2):
    n, d = X.shape
    e = vertex.shape[0]
    mp = 5120
    np_ = 10240
    assert d == 2 * HD and e % CH == 0 and n <= np_
    n_chunks = e // CH

    f32 = jnp.float32
    w1p0 = _padw(W1_w[:, :HD], W)
    w1p1 = _padw(W1_w[:, HD:], W)
    b1p0 = _bias_half(W1_b, 0)
    b1p1 = _bias_half(W1_b, HD)
    w2a = W2_w[:d]
    w2b = W2_w[d:2 * d]
    wr0p0 = _padw(w2b[:HD, :HD], W)
    wr1p0 = _padw(w2b[HD:, :HD], W)
    wr0p1 = _padw(w2b[:HD, HD:], W)
    wr1p1 = _padw(w2b[HD:, HD:], W)
    w2c = W2_w[2 * d:]
    w2cp0 = _padw(w2c[:, :HD], W)
    w2cp1 = _padw(w2c[:, HD:], W)
    b2p0 = _bias_half(W2_b, 0)
    b2p1 = _bias_half(W2_b, HD)
    w3a0 = W3_w1[:HD]
    w3a1 = W3_w1[HD:d]
    w3b = W3_w1[d:2 * d]
    w3c = W3_w1[2 * d:3 * d]
    w3d = W3_w1[3 * d:]

    gpw = -(-n_chunks // NS)
    pad_rows = NS * gpw - n_chunks
    vert2d = jnp.concatenate(
        [vertex.reshape(n_chunks, CH),
         jnp.zeros((pad_rows, CH), jnp.int32)], axis=0)
    edge2d = jnp.concatenate(
        [edges.reshape(n_chunks, CH),
         jnp.zeros((pad_rows, CH), jnp.int32)], axis=0)
    vert_g = jnp.stack([vert2d, vert2d + n])
    edge_g = jnp.stack([edge2d, edge2d + mp])

    bn = 2000
    a2, bp_tab = pl.pallas_call(
        _tc_pre_body,
        grid=(n // bn,),
        in_specs=[
            pl.BlockSpec((bn, d), lambda i: (i, 0)),
            pl.BlockSpec((d, W), lambda i: (0, 0)),
            pl.BlockSpec((d, W), lambda i: (0, 0)),
            pl.BlockSpec((1, W), lambda i: (0, 0)),
            pl.BlockSpec((1, W), lambda i: (0, 0)),
            pl.BlockSpec((d, d), lambda i: (0, 0)),
        ],
        out_specs=[
            pl.BlockSpec((NC, bn, W), lambda i: (0, i, 0)),
            pl.BlockSpec((bn, d), lambda i: (i, 0)),
        ],
        out_shape=[
            jax.ShapeDtypeStruct((NC, n, W), f32),
            jax.ShapeDtypeStruct((n, d), f32),
        ],
    )(X, w1p0, w1p1, b1p0, b1p1, w2a)

    se2 = _sc_segsum(a2.reshape(NC * n, W), vert_g, edge2d, mp, n_chunks)

    bm = 512
    c2 = pl.pallas_call(
        _tc_mid_body,
        grid=(mp // bm,),
        in_specs=[pl.BlockSpec((NC, bm, W), lambda i: (0, i, 0))]
        + [pl.BlockSpec((HD, W), lambda i: (0, 0))] * 4
        + [pl.BlockSpec((1, W), lambda i: (0, 0))] * 4,
        out_specs=pl.BlockSpec((NC, bm, W), lambda i: (0, i, 0)),
        out_shape=jax.ShapeDtypeStruct((NC, mp, W), f32),
    )(se2, wr0p0, wr1p0, wr0p1, wr1p1, w2cp0, w2cp1, b2p0, b2p1)

    t2 = _sc_segsum(c2.reshape(NC * mp, W), edge_g, vert2d, np_, n_chunks)

    bpr = 1280
    out = pl.pallas_call(
        _tc_post_body,
        grid=(np_ // bpr,),
        in_specs=[
            pl.BlockSpec((NC, bpr, W), lambda i: (0, i, 0)),
            pl.BlockSpec((bpr, d), lambda i: (i, 0)),
            pl.BlockSpec((bpr, d), lambda i: (i, 0)),
            pl.BlockSpec((bpr, d), lambda i: (i, 0)),
            pl.BlockSpec((HD, d), lambda i: (0, 0)),
            pl.BlockSpec((HD, d), lambda i: (0, 0)),
            pl.BlockSpec((d, d), lambda i: (0, 0)),
            pl.BlockSpec((d, d), lambda i: (0, 0)),
            pl.BlockSpec((1, d), lambda i: (0, 0)),
            pl.BlockSpec((1, d), lambda i: (0, 0)),
            pl.BlockSpec((d, d), lambda i: (0, 0)),
            pl.BlockSpec((1, d), lambda i: (0, 0)),
        ],
        out_specs=pl.BlockSpec((bpr, d), lambda i: (i, 0)),
        out_shape=jax.ShapeDtypeStruct((n, d), f32),
    )(t2, bp_tab, X, X0, w3a0, w3a1, w3b, w3c, w3d,
      W3_b1.reshape(1, d), W3_w2, W3_b2.reshape(1, d))

    return out


# speedup vs baseline: 1.2461x; 1.2461x over previous
import functools

import jax
import jax.numpy as jnp
from jax import lax
from jax.experimental import pallas as pl
from jax.experimental.pallas import tpu as pltpu
from jax.experimental.pallas import tpu_sc as plsc

NC = 2
NS = 16
LANES = 16
CH = 128
W = 80
HD = 64
NBUF = 2
SK = 3

_DOT = dict(preferred_element_type=jnp.float32, precision=lax.Precision.HIGHEST)


def _padw(m, cols):
    r = m.shape[0]
    return jnp.concatenate(
        [m, jnp.zeros((r, W - m.shape[1]), jnp.float32)], axis=1)


def _bias_half(b, lo):
    return jnp.concatenate(
        [b[lo:lo + HD].reshape(1, HD), jnp.ones((1, 1), jnp.float32),
         jnp.zeros((1, W - HD - 1), jnp.float32)], axis=1)


def _tc_pre_body(x_ref, w1p0_ref, w1p1_ref, b1p0_ref, b1p1_ref, w2a_ref,
                 a_ref, bp_ref):
    x = x_ref[...]
    a_ref[0] = jnp.dot(x, w1p0_ref[...], **_DOT) + b1p0_ref[...]
    a_ref[1] = jnp.dot(x, w1p1_ref[...], **_DOT) + b1p1_ref[...]
    bp_ref[...] = jnp.dot(x, w2a_ref[...], **_DOT)


def _tc_mid_body(se_ref, wr0p0_ref, wr1p0_ref, wr0p1_ref, wr1p1_ref,
                 w2cp0_ref, w2cp1_ref, b2p0_ref, b2p1_ref, c_ref):
    de = jnp.sum(se_ref[0][:, HD:], axis=1)
    inv = 1.0 / jnp.maximum(de, 1.0)
    xe0 = se_ref[0][:, :HD] * inv[:, None]
    xe1 = se_ref[1][:, :HD] * inv[:, None]
    logde = jnp.log(de)[:, None]
    c_ref[0] = (jnp.dot(xe0, wr0p0_ref[...], **_DOT)
                + jnp.dot(xe1, wr1p0_ref[...], **_DOT)
                + logde * w2cp0_ref[...] + b2p0_ref[...])
    c_ref[1] = (jnp.dot(xe0, wr0p1_ref[...], **_DOT)
                + jnp.dot(xe1, wr1p1_ref[...], **_DOT)
                + logde * w2cp1_ref[...] + b2p1_ref[...])


def _tc_post_body(t_ref, bp_ref, x_ref, x0_ref, w3a0_ref, w3a1_ref, w3b_ref,
                  w3c_ref, w3d_ref, b31_ref, w32_ref, b32_ref, out_ref):
    dv = jnp.sum(t_ref[0][:, HD:], axis=1)
    inv = 1.0 / jnp.maximum(dv, 1.0)
    dvc = dv[:, None]
    bp = bp_ref[...]
    xv0 = (dvc * bp[:, :HD] + t_ref[0][:, :HD]) * inv[:, None]
    xv1 = (dvc * bp[:, HD:] + t_ref[1][:, :HD]) * inv[:, None]
    pre = (
        jnp.dot(xv0, w3a0_ref[...], **_DOT)
        + jnp.dot(xv1, w3a1_ref[...], **_DOT)
        + jnp.dot(x_ref[...], w3b_ref[...], **_DOT)
        + jnp.dot(x0_ref[...], w3c_ref[...], **_DOT)
        + jnp.log(dv)[:, None] * w3d_ref[...]
        + b31_ref[...]
    )
    h = jnp.maximum(pre, 0.0)
    out_ref[...] = jnp.dot(h, w32_ref[...], **_DOT) + b32_ref[...]


def _zero_rows(rows_v, n_rows, d):
    zero = jnp.zeros((LANES,), jnp.float32)

    def body(i, carry):
        for c in range(d // LANES):
            rows_v[i, pl.ds(c * LANES, LANES)] = zero
        return carry

    lax.fori_loop(0, n_rows, body, 0)


def _sc_segsum_body(n_chunks, lpw, acc_rows, table_hbm, idx_hbm,
                    out_hbm, acc_sh, idxb, rows_v,
                    semi0, semi1, semi2, semg0, semg1, sems0, sems1):
    cid = lax.axis_index("c")
    tid = lax.axis_index("s")
    semi = (semi0, semi1, semi2)
    semg = (semg0, semg1)
    sems = (sems0, sems1)
    base = tid * lpw

    _zero_rows(rows_v.at[0], CH, W)
    rows_per_tile = acc_rows // NS
    off = 0
    while off < rows_per_tile:
        step = min(CH, rows_per_tile - off)
        pltpu.sync_copy(rows_v.at[0].at[pl.ds(0, step)],
                        acc_sh.at[pl.ds(tid * rows_per_tile + off, step)])
        off += step
    plsc.subcore_barrier()

    def vchunk(k):
        return jnp.logical_and(jnp.logical_and(k >= 0, k < lpw),
                               base + k < n_chunks)

    def idx_start(s, ib):
        @pl.when(vchunk(s * SK))
        def _():
            pltpu.async_copy(idx_hbm.at[cid].at[pl.ds(base + s * SK, SK)],
                             idxb.at[ib], semi[ib])

    def idx_wait(s, ib):
        @pl.when(vchunk(s * SK))
        def _():
            pltpu.make_async_copy(
                idx_hbm.at[cid].at[pl.ds(base + s * SK, SK)],
                idxb.at[ib], semi[ib]).wait()

    def gather_start(s, ib, b):
        for j in range(SK):
            @pl.when(vchunk(s * SK + j))
            def _():
                pltpu.async_copy(table_hbm.at[idxb.at[ib, j, 0]],
                                 rows_v.at[b].at[pl.ds(j * CH, CH)], semg[b])

    def gather_wait(s, ib, b):
        for j in range(SK):
            @pl.when(vchunk(s * SK + j))
            def _():
                pltpu.make_async_copy(
                    table_hbm.at[idxb.at[ib, j, 0]],
                    rows_v.at[b].at[pl.ds(j * CH, CH)], semg[b]).wait()

    def scat_start(s, ib, b):
        for j in range(SK):
            @pl.when(vchunk(s * SK + j))
            def _():
                pltpu.async_copy(rows_v.at[b].at[pl.ds(j * CH, CH)],
                                 acc_sh.at[idxb.at[ib, j, 1]],
                                 sems[b], add=True)

    def scat_wait(s, ib, b):
        for j in range(SK):
            @pl.when(vchunk(s * SK + j))
            def _():
                pltpu.make_async_copy(
                    rows_v.at[b].at[pl.ds(j * CH, CH)],
                    acc_sh.at[idxb.at[ib, j, 1]], sems[b]).wait()

    n_super = -(-lpw // SK)
    idx_start(0, 0)
    idx_start(1, 1)
    idx_wait(0, 0)
    gather_start(0, 0, 0)

    def ring(s6, carry):
        for u in range(6):
            s = s6 * 6 + u
            b, nb = u % 2, (u + 1) % 2
            ib, ibn, ibp = u % 3, (u + 1) % 3, (u + 2) % 3
            idx_wait(s + 1, ibn)
            gather_wait(s, ib, b)
            scat_start(s, ib, b)
            scat_wait(s - 1, ibp, nb)
            gather_start(s + 1, ibn, nb)
            idx_start(s + 2, ibp)
        return carry

    lax.fori_loop(0, -(-(n_super + 1) // 6), ring, 0)
    plsc.subcore_barrier()

    pltpu.sync_copy(acc_sh.at[pl.ds(tid * rows_per_tile, rows_per_tile)],
                    out_hbm.at[cid, pl.ds(tid * rows_per_tile, rows_per_tile)])


def _sc_segsum(table_flat, idx4d, acc_rows, n_chunks, lpw):
    mesh = plsc.VectorSubcoreMesh(core_axis_name="c", subcore_axis_name="s")
    return pl.kernel(
        functools.partial(_sc_segsum_body, n_chunks, lpw, acc_rows),
        out_type=jax.ShapeDtypeStruct((NC, acc_rows, W), jnp.float32),
        mesh=mesh,
        scratch_types=[
            pltpu.VMEM_SHARED((acc_rows, W), jnp.float32),
            pltpu.VMEM((3, SK, 2, CH), jnp.int32),
            pltpu.VMEM((NBUF, SK * CH, W), jnp.float32),
        ] + [pltpu.SemaphoreType.DMA] * 7,
        compiler_params=pltpu.CompilerParams(use_tc_tiling_on_sc=False),
    )(table_flat, idx4d)


def kernel(X, vertex, edges, X0, W1_w, W1_b, W2_w, W2_b, W3_w1, W3_b1,
           W3_w2, W3_b2):
    n, d = X.shape
    e = vertex.shape[0]
    mp = 5120
    np_ = 10240
    assert d == 2 * HD and e % CH == 0 and n <= np_
    n_chunks = e // CH

    f32 = jnp.float32
    w1p0 = _padw(W1_w[:, :HD], W)
    w1p1 = _padw(W1_w[:, HD:], W)
    b1p0 = _bias_half(W1_b, 0)
    b1p1 = _bias_half(W1_b, HD)
    w2a = W2_w[:d]
    w2b = W2_w[d:2 * d]
    wr0p0 = _padw(w2b[:HD, :HD], W)
    wr1p0 = _padw(w2b[HD:, :HD], W)
    wr0p1 = _padw(w2b[:HD, HD:], W)
    wr1p1 = _padw(w2b[HD:, HD:], W)
    w2c = W2_w[2 * d:]
    w2cp0 = _padw(w2c[:, :HD], W)
    w2cp1 = _padw(w2c[:, HD:], W)
    b2p0 = _bias_half(W2_b, 0)
    b2p1 = _bias_half(W2_b, HD)
    w3a0 = W3_w1[:HD]
    w3a1 = W3_w1[HD:d]
    w3b = W3_w1[d:2 * d]
    w3c = W3_w1[2 * d:3 * d]
    w3d = W3_w1[3 * d:]

    lpw = SK * (-(-(-(-n_chunks // NS)) // SK))
    pad_rows = NS * lpw - n_chunks
    vert2d = jnp.concatenate(
        [vertex.reshape(n_chunks, CH),
         jnp.zeros((pad_rows, CH), jnp.int32)], axis=0)
    edge2d = jnp.concatenate(
        [edges.reshape(n_chunks, CH),
         jnp.zeros((pad_rows, CH), jnp.int32)], axis=0)
    idx_p1 = jnp.stack([jnp.stack([vert2d, edge2d], axis=1),
                        jnp.stack([vert2d + n, edge2d], axis=1)])
    idx_p2 = jnp.stack([jnp.stack([edge2d, vert2d], axis=1),
                        jnp.stack([edge2d + mp, vert2d], axis=1)])

    bn = 2000
    a2, bp_tab = pl.pallas_call(
        _tc_pre_body,
        grid=(n // bn,),
        in_specs=[
            pl.BlockSpec((bn, d), lambda i: (i, 0)),
            pl.BlockSpec((d, W), lambda i: (0, 0)),
            pl.BlockSpec((d, W), lambda i: (0, 0)),
            pl.BlockSpec((1, W), lambda i: (0, 0)),
            pl.BlockSpec((1, W), lambda i: (0, 0)),
            pl.BlockSpec((d, d), lambda i: (0, 0)),
        ],
        out_specs=[
            pl.BlockSpec((NC, bn, W), lambda i: (0, i, 0)),
            pl.BlockSpec((bn, d), lambda i: (i, 0)),
        ],
        out_shape=[
            jax.ShapeDtypeStruct((NC, n, W), f32),
            jax.ShapeDtypeStruct((n, d), f32),
        ],
    )(X, w1p0, w1p1, b1p0, b1p1, w2a)

    se2 = _sc_segsum(a2.reshape(NC * n, W), idx_p1, mp, n_chunks, lpw)

    bm = 512
    c2 = pl.pallas_call(
        _tc_mid_body,
        grid=(mp // bm,),
        in_specs=[pl.BlockSpec((NC, bm, W), lambda i: (0, i, 0))]
        + [pl.BlockSpec((HD, W), lambda i: (0, 0))] * 4
        + [pl.BlockSpec((1, W), lambda i: (0, 0))] * 4,
        out_specs=pl.BlockSpec((NC, bm, W), lambda i: (0, i, 0)),
        out_shape=jax.ShapeDtypeStruct((NC, mp, W), f32),
    )(se2, wr0p0, wr1p0, wr0p1, wr1p1, w2cp0, w2cp1, b2p0, b2p1)

    t2 = _sc_segsum(c2.reshape(NC * mp, W), idx_p2, np_, n_chunks, lpw)

    bpr = 1280
    out = pl.pallas_call(
        _tc_post_body,
        grid=(np_ // bpr,),
        in_specs=[
            pl.BlockSpec((NC, bpr, W), lambda i: (0, i, 0)),
            pl.BlockSpec((bpr, d), lambda i: (i, 0)),
            pl.BlockSpec((bpr, d), lambda i: (i, 0)),
            pl.BlockSpec((bpr, d), lambda i: (i, 0)),
            pl.BlockSpec((HD, d), lambda i: (0, 0)),
            pl.BlockSpec((HD, d), lambda i: (0, 0)),
            pl.BlockSpec((d, d), lambda i: (0, 0)),
            pl.BlockSpec((d, d), lambda i: (0, 0)),
            pl.BlockSpec((1, d), lambda i: (0, 0)),
            pl.BlockSpec((1, d), lambda i: (0, 0)),
            pl.BlockSpec((d, d), lambda i: (0, 0)),
            pl.BlockSpec((1, d), lambda i: (0, 0)),
        ],
        out_specs=pl.BlockSpec((bpr, d), lambda i: (i, 0)),
        out_shape=jax.ShapeDtypeStruct((n, d), f32),
    )(t2, bp_tab, X, X0, w3a0, w3a1, w3b, w3c, w3d,
      W3_b1.reshape(1, d), W3_w2, W3_b2.reshape(1, d))

    return out


# speedup vs baseline: 1.3747x; 1.1032x over previous
import functools

import jax
import jax.numpy as jnp
from jax import lax
from jax.experimental import pallas as pl
from jax.experimental.pallas import tpu as pltpu
from jax.experimental.pallas import tpu_sc as plsc

NC = 2
NS = 16
LANES = 16
CH = 128
HD = 64
NBUF = 3
SK = 3
NIB = 4

_DOT = dict(preferred_element_type=jnp.float32, precision=lax.Precision.HIGHEST)


def _tc_pre_body(x_ref, w10_ref, w11_ref, b10_ref, b11_ref, w2a_ref,
                 a_ref, bp_ref):
    x = x_ref[...]
    a_ref[0] = jnp.dot(x, w10_ref[...], **_DOT) + b10_ref[...]
    a_ref[1] = jnp.dot(x, w11_ref[...], **_DOT) + b11_ref[...]
    bp_ref[...] = jnp.dot(x, w2a_ref[...], **_DOT)


def _tc_mid_body(se_ref, dege_ref, wr00_ref, wr10_ref, wr01_ref, wr11_ref,
                 w2c0_ref, w2c1_ref, b20_ref, b21_ref, c_ref):
    de = jnp.sum(dege_ref[...], axis=(0, 2))
    inv = 1.0 / jnp.maximum(de, 1.0)
    xe0 = se_ref[0] * inv[:, None]
    xe1 = se_ref[1] * inv[:, None]
    logde = jnp.log(de)[:, None]
    c_ref[0] = (jnp.dot(xe0, wr00_ref[...], **_DOT)
                + jnp.dot(xe1, wr10_ref[...], **_DOT)
                + logde * w2c0_ref[...] + b20_ref[...])
    c_ref[1] = (jnp.dot(xe0, wr01_ref[...], **_DOT)
                + jnp.dot(xe1, wr11_ref[...], **_DOT)
                + logde * w2c1_ref[...] + b21_ref[...])


def _tc_post_body(t_ref, degv_ref, bp_ref, x_ref, x0_ref, w3a0_ref, w3a1_ref,
                  w3b_ref, w3c_ref, w3d_ref, b31_ref, w32_ref, b32_ref,
                  out_ref):
    dv = jnp.sum(degv_ref[...], axis=(0, 2))
    inv = 1.0 / jnp.maximum(dv, 1.0)
    dvc = dv[:, None]
    bp = bp_ref[...]
    xv0 = (dvc * bp[:, :HD] + t_ref[0]) * inv[:, None]
    xv1 = (dvc * bp[:, HD:] + t_ref[1]) * inv[:, None]
    pre = (
        jnp.dot(xv0, w3a0_ref[...], **_DOT)
        + jnp.dot(xv1, w3a1_ref[...], **_DOT)
        + jnp.dot(x_ref[...], w3b_ref[...], **_DOT)
        + jnp.dot(x0_ref[...], w3c_ref[...], **_DOT)
        + jnp.log(dv)[:, None] * w3d_ref[...]
        + b31_ref[...]
    )
    h = jnp.maximum(pre, 0.0)
    out_ref[...] = jnp.dot(h, w32_ref[...], **_DOT) + b32_ref[...]


def _zero_rows(rows_v, n_rows, d):
    zero = jnp.zeros((LANES,), jnp.float32)

    def body(i, carry):
        for c in range(d // LANES):
            rows_v[i, pl.ds(c * LANES, LANES)] = zero
        return carry

    lax.fori_loop(0, n_rows, body, 0)


def _sc_segsum_body(n_chunks, lpw, acc_rows, tab_half, do_hist,
                    *refs):
    if do_hist:
        (table_hbm, idx_hbm, out_hbm, dege_hbm, degv_hbm,
         acc_sh, idxb, rows_v, dege_sh, degv_sh, ones_b, zero_b,
         *sems) = refs
    else:
        (table_hbm, idx_hbm, out_hbm,
         acc_sh, idxb, rows_v, *sems) = refs
    semi = sems[:NIB]
    semg = sems[NIB:NIB + NBUF]
    sems_s = sems[NIB + NBUF:]
    cid = lax.axis_index("c")
    tid = lax.axis_index("s")
    base = tid * lpw

    _zero_rows(rows_v.at[0], CH, HD)
    rows_per_tile = acc_rows // NS
    off = 0
    while off < rows_per_tile:
        step = min(CH, rows_per_tile - off)
        pltpu.sync_copy(rows_v.at[0].at[pl.ds(0, step)],
                        acc_sh.at[pl.ds(tid * rows_per_tile + off, step)])
        off += step
    if do_hist:
        one0 = jnp.where(lax.broadcasted_iota(jnp.int32, (LANES,), 0) == 0,
                         1.0, 0.0)
        zero = jnp.zeros((LANES,), jnp.float32)

        def fill(i, carry):
            ones_b[i, :] = one0
            zero_b[i, :] = zero
            return carry

        lax.fori_loop(0, CH, fill, 0)
        for h_sh in (dege_sh, degv_sh):
            rpt = h_sh.shape[0] // NS
            off = 0
            while off < rpt:
                step = min(CH, rpt - off)
                pltpu.sync_copy(zero_b.at[pl.ds(0, step)],
                                h_sh.at[pl.ds(tid * rpt + off, step)])
                off += step
    plsc.subcore_barrier()

    def vchunk(k):
        return jnp.logical_and(jnp.logical_and(k >= 0, k < lpw),
                               base + k < n_chunks)

    def idx_start(s, i):
        @pl.when(vchunk(s * SK))
        def _():
            pltpu.async_copy(idx_hbm.at[cid].at[pl.ds(base + s * SK, SK)],
                             idxb.at[i], semi[i])

    def idx_wait(s, i):
        @pl.when(vchunk(s * SK))
        def _():
            pltpu.make_async_copy(
                idx_hbm.at[cid].at[pl.ds(base + s * SK, SK)],
                idxb.at[i], semi[i]).wait()

    def gather_start(s, i, b):
        for j in range(SK):
            @pl.when(vchunk(s * SK + j))
            def _():
                pltpu.async_copy(table_hbm.at[idxb.at[i, j, 0]],
                                 rows_v.at[b].at[pl.ds(j * CH, CH)], semg[b])

    def gather_wait(s, i, b):
        for j in range(SK):
            @pl.when(vchunk(s * SK + j))
            def _():
                pltpu.make_async_copy(
                    table_hbm.at[idxb.at[i, j, 0]],
                    rows_v.at[b].at[pl.ds(j * CH, CH)], semg[b]).wait()

    def scat_start(s, i, b):
        for j in range(SK):
            @pl.when(vchunk(s * SK + j))
            def _():
                pltpu.async_copy(rows_v.at[b].at[pl.ds(j * CH, CH)],
                                 acc_sh.at[idxb.at[i, j, 1]],
                                 sems_s[b], add=True)
                if do_hist:
                    @pl.when(cid == 0)
                    def _():
                        pltpu.async_copy(ones_b,
                                         dege_sh.at[idxb.at[i, j, 2]],
                                         sems_s[b], add=True)

                    @pl.when(cid == 1)
                    def _():
                        pltpu.async_copy(ones_b,
                                         degv_sh.at[idxb.at[i, j, 2]],
                                         sems_s[b], add=True)

    def scat_wait(s, i, b):
        for j in range(SK):
            @pl.when(vchunk(s * SK + j))
            def _():
                pltpu.make_async_copy(
                    rows_v.at[b].at[pl.ds(j * CH, CH)],
                    acc_sh.at[idxb.at[i, j, 1]], sems_s[b]).wait()
                if do_hist:
                    @pl.when(cid == 0)
                    def _():
                        pltpu.make_async_copy(
                            ones_b, dege_sh.at[idxb.at[i, j, 2]],
                            sems_s[b]).wait()

                    @pl.when(cid == 1)
                    def _():
                        pltpu.make_async_copy(
                            ones_b, degv_sh.at[idxb.at[i, j, 2]],
                            sems_s[b]).wait()

    n_super = lpw // SK
    idx_start(0, 0)
    idx_start(1, 1)
    idx_wait(0, 0)
    gather_start(0, 0, 0)

    def ring(s12, carry):
        for u in range(12):
            s = s12 * 12 + u
            b, bn = u % NBUF, (u + 1) % NBUF
            i, i1, i2 = u % NIB, (u + 1) % NIB, (u + 2) % NIB
            idx_wait(s + 1, i1)
            gather_wait(s, i, b)
            scat_start(s, i, b)
            scat_wait(s - 2, i2, bn)
            gather_start(s + 1, i1, bn)
            idx_start(s + 2, i2)
        return carry

    lax.fori_loop(0, -(-(n_super + 2) // 12), ring, 0)
    plsc.subcore_barrier()

    pltpu.sync_copy(acc_sh.at[pl.ds(tid * rows_per_tile, rows_per_tile)],
                    out_hbm.at[cid, pl.ds(tid * rows_per_tile, rows_per_tile)])
    if do_hist:
        rpt_e = dege_sh.shape[0] // NS
        pltpu.sync_copy(dege_sh.at[pl.ds(tid * rpt_e, rpt_e)],
                        dege_hbm.at[cid, pl.ds(tid * rpt_e, rpt_e)])
        rpt_v = degv_sh.shape[0] // NS
        pltpu.sync_copy(degv_sh.at[pl.ds(tid * rpt_v, rpt_v)],
                        degv_hbm.at[cid, pl.ds(tid * rpt_v, rpt_v)])


def _sc_segsum(table_flat, idx4d, acc_rows, n_chunks, lpw, hist_sizes=None):
    mesh = plsc.VectorSubcoreMesh(core_axis_name="c", subcore_axis_name="s")
    do_hist = hist_sizes is not None
    tab_half = table_flat.shape[0] // NC
    ncols = idx4d.shape[2]
    out_type = [jax.ShapeDtypeStruct((NC, acc_rows, HD), jnp.float32)]
    scratch = [
        pltpu.VMEM_SHARED((acc_rows, HD), jnp.float32),
        pltpu.VMEM((NIB, SK, ncols, CH), jnp.int32),
        pltpu.VMEM((NBUF, SK * CH, HD), jnp.float32),
    ]
    if do_hist:
        me, nv = hist_sizes
        out_type += [jax.ShapeDtypeStruct((NC, me, LANES), jnp.float32),
                     jax.ShapeDtypeStruct((NC, nv, LANES), jnp.float32)]
        scratch += [pltpu.VMEM_SHARED((me, LANES), jnp.float32),
                    pltpu.VMEM_SHARED((nv, LANES), jnp.float32),
                    pltpu.VMEM((CH, LANES), jnp.float32),
                    pltpu.VMEM((CH, LANES), jnp.float32)]
    res = pl.kernel(
        functools.partial(_sc_segsum_body, n_chunks, lpw, acc_rows,
                          tab_half, do_hist),
        out_type=out_type,
        mesh=mesh,
        scratch_types=scratch + [pltpu.SemaphoreType.DMA] * (NIB + 2 * NBUF),
        compiler_params=pltpu.CompilerParams(use_tc_tiling_on_sc=False),
    )(table_flat, idx4d)
    return res


def kernel(X, vertex, edges, X0, W1_w, W1_b, W2_w, W2_b, W3_w1, W3_b1,
           W3_w2, W3_b2):
    n, d = X.shape
    e = vertex.shape[0]
    mp = 5120
    np_ = 10240
    assert d == 2 * HD and e % CH == 0 and n <= np_
    n_chunks = e // CH

    f32 = jnp.float32
    w10 = W1_w[:, :HD]
    w11 = W1_w[:, HD:]
    b10 = W1_b[:HD].reshape(1, HD)
    b11 = W1_b[HD:].reshape(1, HD)
    w2a = W2_w[:d]
    w2b = W2_w[d:2 * d]
    wr00 = w2b[:HD, :HD]
    wr10 = w2b[HD:, :HD]
    wr01 = w2b[:HD, HD:]
    wr11 = w2b[HD:, HD:]
    w2c = W2_w[2 * d:]
    w2c0 = w2c[:, :HD]
    w2c1 = w2c[:, HD:]
    b20 = W2_b[:HD].reshape(1, HD)
    b21 = W2_b[HD:].reshape(1, HD)
    w3a0 = W3_w1[:HD]
    w3a1 = W3_w1[HD:d]
    w3b = W3_w1[d:2 * d]
    w3c = W3_w1[2 * d:3 * d]
    w3d = W3_w1[3 * d:]

    lpw = SK * (-(-(-(-n_chunks // NS)) // SK))
    pad_rows = NS * lpw - n_chunks
    vert2d = jnp.concatenate(
        [vertex.reshape(n_chunks, CH),
         jnp.zeros((pad_rows, CH), jnp.int32)], axis=0)
    edge2d = jnp.concatenate(
        [edges.reshape(n_chunks, CH),
         jnp.zeros((pad_rows, CH), jnp.int32)], axis=0)
    idx_p1 = jnp.stack([jnp.stack([vert2d, edge2d, edge2d], axis=1),
                        jnp.stack([vert2d + n, edge2d, vert2d], axis=1)])
    idx_p2 = jnp.stack([jnp.stack([edge2d, vert2d], axis=1),
                        jnp.stack([edge2d + mp, vert2d], axis=1)])

    bn = 2000
    a2, bp_tab = pl.pallas_call(
        _tc_pre_body,
        grid=(n // bn,),
        in_specs=[
            pl.BlockSpec((bn, d), lambda i: (i, 0)),
            pl.BlockSpec((d, HD), lambda i: (0, 0)),
            pl.BlockSpec((d, HD), lambda i: (0, 0)),
            pl.BlockSpec((1, HD), lambda i: (0, 0)),
            pl.BlockSpec((1, HD), lambda i: (0, 0)),
            pl.BlockSpec((d, d), lambda i: (0, 0)),
        ],
        out_specs=[
            pl.BlockSpec((NC, bn, HD), lambda i: (0, i, 0)),
            pl.BlockSpec((bn, d), lambda i: (i, 0)),
        ],
        out_shape=[
            jax.ShapeDtypeStruct((NC, n, HD), f32),
            jax.ShapeDtypeStruct((n, d), f32),
        ],
    )(X, w10, w11, b10, b11, w2a)

    se2, dege_p, degv_p = _sc_segsum(
        a2.reshape(NC * n, HD), idx_p1, mp, n_chunks, lpw,
        hist_sizes=(mp, np_))

    bm = 512
    c2 = pl.pallas_call(
        _tc_mid_body,
        grid=(mp // bm,),
        in_specs=[pl.BlockSpec((NC, bm, HD), lambda i: (0, i, 0)),
                  pl.BlockSpec((NC, bm, LANES), lambda i: (0, i, 0))]
        + [pl.BlockSpec((HD, HD), lambda i: (0, 0))] * 4
        + [pl.BlockSpec((1, HD), lambda i: (0, 0))] * 4,
        out_specs=pl.BlockSpec((NC, bm, HD), lambda i: (0, i, 0)),
        out_shape=jax.ShapeDtypeStruct((NC, mp, HD), f32),
    )(se2, dege_p, wr00, wr10, wr01, wr11, w2c0, w2c1, b20, b21)

    (t2,) = _sc_segsum(c2.reshape(NC * mp, HD), idx_p2, np_, n_chunks, lpw)

    bpr = 1280
    out = pl.pallas_call(
        _tc_post_body,
        grid=(np_ // bpr,),
        in_specs=[
            pl.BlockSpec((NC, bpr, HD), lambda i: (0, i, 0)),
            pl.BlockSpec((NC, bpr, LANES), lambda i: (0, i, 0)),
            pl.BlockSpec((bpr, d), lambda i: (i, 0)),
            pl.BlockSpec((bpr, d), lambda i: (i, 0)),
            pl.BlockSpec((bpr, d), lambda i: (i, 0)),
            pl.BlockSpec((HD, d), lambda i: (0, 0)),
            pl.BlockSpec((HD, d), lambda i: (0, 0)),
            pl.BlockSpec((d, d), lambda i: (0, 0)),
            pl.BlockSpec((d, d), lambda i: (0, 0)),
            pl.BlockSpec((1, d), lambda i: (0, 0)),
            pl.BlockSpec((1, d), lambda i: (0, 0)),
            pl.BlockSpec((d, d), lambda i: (0, 0)),
            pl.BlockSpec((1, d), lambda i: (0, 0)),
        ],
        out_specs=pl.BlockSpec((bpr, d), lambda i: (i, 0)),
        out_shape=jax.ShapeDtypeStruct((n, d), f32),
    )(t2, degv_p, bp_tab, X, X0, w3a0, w3a1, w3b, w3c, w3d,
      W3_b1.reshape(1, d), W3_w2, W3_b2.reshape(1, d))

    return out
